# baseline (speedup 1.0000x reference)
import math

import jax
import jax.numpy as jnp
from jax import lax
from jax.experimental import pallas as pl
from jax.experimental.pallas import tpu as pltpu
from jax.experimental.pallas import tpu_sc as plsc

B = 1024
V = 100000
S = 16.0
SHIFT = 16.0
M_MARGIN = 0.1
COS_M = math.cos(M_MARGIN)
SIN_M = math.sin(M_MARGIN)
COS_PI_M = math.cos(math.pi - M_MARGIN)
SIN_PI_M = math.sin(math.pi - M_MARGIN)

NC = 2
NS = 16
L = 16
NW = NC * NS
BPW = B // NW

TAIL0 = (V // 128) * 128
CB_MAX = TAIL0 - 128

RB = 32
TC_GRID = B // RB


def _sc_body(in_hbm, tail_hbm, tgt_hbm, ct_hbm, idx_v, win_v, tail_v, val_v, sem):
    wid = lax.axis_index("s") * NC + lax.axis_index("c")
    base = wid * BPW
    pltpu.sync_copy(tgt_hbm.at[pl.ds(base, BPW)], idx_v)
    copies = []
    for rg in range(BPW // 8):
        r0 = pl.multiple_of(base + rg * 8, 8)
        copies.append(
            pltpu.async_copy(
                tail_hbm.at[pl.ds(r0, 8), :], tail_v.at[pl.ds(rg * 8, 8), :], sem
            )
        )
    for g in range(BPW // L):
        cvec = idx_v[pl.ds(g * L, L)]
        cbvec = jnp.minimum((cvec // 128) * 128, CB_MAX)
        for j in range(L):
            i = g * L + j
            r0 = pl.multiple_of(base + (i // 8) * 8, 8)
            cb = pl.multiple_of(cbvec[j], 128)
            copies.append(
                pltpu.async_copy(
                    in_hbm.at[pl.ds(r0, 8), pl.ds(cb, 128)], win_v.at[i], sem
                )
            )
    for cp in copies:
        cp.wait()
    for g in range(BPW // L):
        i16 = g * L + lax.iota(jnp.int32, L)
        cvec = idx_v[pl.ds(g * L, L)]
        cbvec = jnp.minimum((cvec // 128) * 128, CB_MAX)
        rin = lax.rem(i16, 8)
        main_off = jnp.minimum(jnp.maximum(cvec - cbvec, 0), 127)
        v_main = plsc.load_gather(win_v, [i16, rin, main_off])
        tail_off = jnp.minimum(jnp.maximum(cvec - TAIL0, 0), 127)
        v_tail = plsc.load_gather(tail_v, [i16, tail_off])
        val_v[pl.ds(g * L, L)] = jnp.where(cvec >= TAIL0, v_tail, v_main)
    pltpu.sync_copy(val_v, ct_hbm.at[pl.ds(base, BPW)])


def _sc_gather(input, tail, target):
    mesh = plsc.VectorSubcoreMesh(core_axis_name="c", subcore_axis_name="s")
    return pl.kernel(
        _sc_body,
        mesh=mesh,
        compiler_params=pltpu.CompilerParams(needs_layout_passes=False),
        out_type=jax.ShapeDtypeStruct((B,), jnp.float32),
        scratch_types=[
            pltpu.VMEM((BPW,), jnp.int32),
            pltpu.VMEM((BPW, 8, 128), jnp.float32),
            pltpu.VMEM((BPW, 128), jnp.float32),
            pltpu.VMEM((BPW,), jnp.float32),
            pltpu.SemaphoreType.DMA,
        ],
    )(input, tail, target)


def _lse_body(in_ref, tl_ref, ct_ref, out_ref, s_scr):
    i = pl.program_id(0)
    part = jnp.sum(jnp.exp(in_ref[...] * S - SHIFT), axis=1, keepdims=True)
    part += jnp.sum(jnp.exp(tl_ref[...] * S - SHIFT), axis=1, keepdims=True)
    s_scr[pl.ds(i * RB, RB), :] = part

    @pl.when(i == TC_GRID - 1)
    def _combine():
        ssum = s_scr[...]
        ct = ct_ref[...]
        sin_t = jnp.sqrt(1.0 - ct * ct)
        phi = ct * COS_M - sin_t * SIN_M
        keep = ct - SIN_PI_M * M_MARGIN
        new_val = jnp.where(ct - COS_PI_M > 0, phi, keep)
        s_adj = ssum - jnp.exp(S * ct - SHIFT) + jnp.exp(S * new_val - SHIFT)
        logz = SHIFT + jnp.log(s_adj)
        nll = logz - S * new_val
        out_ref[...] = jnp.sum(nll, keepdims=True).reshape(1, 1) / B


def _tc_lse_loss(input, tail, cos_t):
    return pl.pallas_call(
        _lse_body,
        grid=(TC_GRID,),
        in_specs=[
            pl.BlockSpec((RB, TAIL0), lambda i: (i, 0)),
            pl.BlockSpec((RB, 128), lambda i: (i, 0)),
            pl.BlockSpec((B, 1), lambda i: (0, 0)),
        ],
        out_specs=pl.BlockSpec((1, 1), lambda i: (0, 0)),
        out_shape=jax.ShapeDtypeStruct((1, 1), jnp.float32),
        scratch_shapes=[pltpu.VMEM((B, 1), jnp.float32)],
    )(input, tail, cos_t)


@jax.jit
def kernel(input, target):
    target = target.astype(jnp.int32)
    tail = jnp.pad(
        input[:, TAIL0:], ((0, 0), (0, 128 - (V - TAIL0))), constant_values=-1000.0
    )
    cos_t = _sc_gather(input, tail, target)
    loss = _tc_lse_loss(input, tail, cos_t.reshape(B, 1))
    return loss[0, 0]


# speedup vs baseline: 1.0121x; 1.0121x over previous
import math

import jax
import jax.numpy as jnp
from jax import lax
from jax.experimental import pallas as pl
from jax.experimental.pallas import tpu as pltpu
from jax.experimental.pallas import tpu_sc as plsc

B = 1024
V = 100000
S = 16.0
SHIFT = 16.0
M_MARGIN = 0.1
COS_M = math.cos(M_MARGIN)
SIN_M = math.sin(M_MARGIN)
COS_PI_M = math.cos(math.pi - M_MARGIN)
SIN_PI_M = math.sin(math.pi - M_MARGIN)

NC = 2
NS = 16
L = 16
NW = NC * NS
BPW = B // NW

TAIL0 = (V // 128) * 128
CB_MAX = TAIL0 - 128

RB = 32
TC_GRID = B // RB


def _sc_body(in_hbm, tail_hbm, tgt_hbm, ct_hbm, ps_hbm,
             idx_v, win_v, tail_v, val_v, acc_v, sem):
    wid = lax.axis_index("s") * NC + lax.axis_index("c")
    base = wid * BPW
    pltpu.sync_copy(tgt_hbm.at[pl.ds(base, BPW)], idx_v)
    copies = []
    for rg in range(BPW // 8):
        r0 = pl.multiple_of(base + rg * 8, 8)
        copies.append(
            pltpu.async_copy(
                tail_hbm.at[pl.ds(r0, 8), :], tail_v.at[pl.ds(rg * 8, 8), :], sem
            )
        )
    for g in range(BPW // L):
        cvec = idx_v[pl.ds(g * L, L)]
        cbvec = jnp.minimum((cvec // 128) * 128, CB_MAX)
        for j in range(L):
            i = g * L + j
            r0 = pl.multiple_of(base + (i // 8) * 8, 8)
            cb = pl.multiple_of(cbvec[j], 128)
            copies.append(
                pltpu.async_copy(
                    in_hbm.at[pl.ds(r0, 8), pl.ds(cb, 128)], win_v.at[i], sem
                )
            )
    for cp in copies:
        cp.wait()
    for g in range(BPW // L):
        i16 = g * L + lax.iota(jnp.int32, L)
        cvec = idx_v[pl.ds(g * L, L)]
        cbvec = jnp.minimum((cvec // 128) * 128, CB_MAX)
        rin = lax.rem(i16, 8)
        main_off = jnp.minimum(jnp.maximum(cvec - cbvec, 0), 127)
        v_main = plsc.load_gather(win_v, [i16, rin, main_off])
        tail_off = jnp.minimum(jnp.maximum(cvec - TAIL0, 0), 127)
        v_tail = plsc.load_gather(tail_v, [i16, tail_off])
        val_v[pl.ds(g * L, L)] = jnp.where(cvec >= TAIL0, v_tail, v_main)
    pltpu.sync_copy(val_v, ct_hbm.at[pl.ds(base, BPW)])

    for rg in range(BPW // 8):
        acc = [jnp.zeros((L,), jnp.float32) for _ in range(8)]
        for r in range(8):
            for t in range(128 // L):
                acc[r] = acc[r] + jnp.exp(
                    tail_v[rg * 8 + r, pl.ds(t * L, L)] * S - SHIFT
                )
        for r in range(8):
            acc_v[rg * 8 + r, :] = acc[r]
    pltpu.sync_copy(acc_v, ps_hbm.at[pl.ds(base, BPW)])


def _sc_gather_and_partials(input, tail, target):
    mesh = plsc.VectorSubcoreMesh(core_axis_name="c", subcore_axis_name="s")
    return pl.kernel(
        _sc_body,
        mesh=mesh,
        compiler_params=pltpu.CompilerParams(needs_layout_passes=False),
        out_type=[
            jax.ShapeDtypeStruct((B,), jnp.float32),
            jax.ShapeDtypeStruct((B, L), jnp.float32),
        ],
        scratch_types=[
            pltpu.VMEM((BPW,), jnp.int32),
            pltpu.VMEM((BPW, 8, 128), jnp.float32),
            pltpu.VMEM((BPW, 128), jnp.float32),
            pltpu.VMEM((BPW,), jnp.float32),
            pltpu.VMEM((BPW, L), jnp.float32),
            pltpu.SemaphoreType.DMA,
        ],
    )(input, tail, target)


def _lse_body(in_ref, s_out):
    s_out[...] = jnp.sum(
        jnp.exp(in_ref[...] * S - SHIFT), axis=1, keepdims=True
    )


def _tc_lse(input):
    return pl.pallas_call(
        _lse_body,
        grid=(TC_GRID,),
        in_specs=[pl.BlockSpec((RB, TAIL0), lambda i: (i, 0))],
        out_specs=pl.BlockSpec((RB, 1), lambda i: (i, 0)),
        out_shape=jax.ShapeDtypeStruct((B, 1), jnp.float32),
    )(input)


def _combine_body(s_ref, ps_ref, ct_ref, out_ref):
    ssum = s_ref[...] + jnp.sum(ps_ref[...], axis=1, keepdims=True)
    ct = ct_ref[...]
    sin_t = jnp.sqrt(1.0 - ct * ct)
    phi = ct * COS_M - sin_t * SIN_M
    keep = ct - SIN_PI_M * M_MARGIN
    new_val = jnp.where(ct - COS_PI_M > 0, phi, keep)
    s_adj = ssum - jnp.exp(S * ct - SHIFT) + jnp.exp(S * new_val - SHIFT)
    logz = SHIFT + jnp.log(s_adj)
    nll = logz - S * new_val
    out_ref[...] = jnp.sum(nll, keepdims=True).reshape(1, 1) / B


def _tc_combine(ssum, partials, cos_t):
    return pl.pallas_call(
        _combine_body,
        out_shape=jax.ShapeDtypeStruct((1, 1), jnp.float32),
    )(ssum, partials, cos_t)


@jax.jit
def kernel(input, target):
    target = target.astype(jnp.int32)
    tail = jnp.pad(
        input[:, TAIL0:], ((0, 0), (0, 128 - (V - TAIL0))), constant_values=-1000.0
    )
    ssum = _tc_lse(input)
    cos_t, partials = _sc_gather_and_partials(input, tail, target)
    loss = _tc_combine(ssum, partials, cos_t.reshape(B, 1))
    return loss[0, 0]
